# Initial kernel scaffold; baseline (speedup 1.0000x reference)
import jax
import jax.numpy as jnp
from jax.experimental import pallas as pl


def kernel(z, embeddings):
    raise NotImplementedError("write your pallas kernel here")


# speedup vs baseline: 1.0664x; 1.0664x over previous
import functools

import jax
import jax.numpy as jnp
from jax import lax
from jax.experimental import pallas as pl
from jax.experimental.pallas import tpu as pltpu
from jax.experimental.pallas import tpu_sc as plsc

K_TOTAL = 8192
D_DIM = 256
K_CHUNK = 1024


def _argmin_body(z_ref, emb_ref, out_ref):
    zb = z_ref[0]
    T = zb.shape[1]
    z2 = jnp.sum(zb * zb, axis=0, keepdims=True)

    def step(c, carry):
        minv, mini = carry
        ech = emb_ref[pl.ds(c * K_CHUNK, K_CHUNK), :]
        e2 = jnp.sum(ech * ech, axis=1, keepdims=True)
        m = jnp.dot(ech, zb, preferred_element_type=jnp.float32)
        d = (z2 + e2) - 2.0 * m
        cmin = jnp.min(d, axis=0, keepdims=True)
        ids = lax.broadcasted_iota(jnp.int32, d.shape, 0) + c * K_CHUNK
        cidx = jnp.min(
            jnp.where(d == cmin, ids, jnp.int32(2**30)), axis=0, keepdims=True
        )
        better = cmin < minv
        return (jnp.where(better, cmin, minv), jnp.where(better, cidx, mini))

    minv0 = jnp.full((1, T), jnp.inf, dtype=jnp.float32)
    mini0 = jnp.zeros((1, T), dtype=jnp.int32)
    _, mini = lax.fori_loop(0, K_TOTAL // K_CHUNK, step, (minv0, mini0))
    out_ref[0] = mini


def _tc_argmin(z, embeddings):
    B, D, T = z.shape
    return pl.pallas_call(
        _argmin_body,
        grid=(B,),
        in_specs=[
            pl.BlockSpec((1, D, T), lambda b: (b, 0, 0)),
            pl.BlockSpec((K_TOTAL, D), lambda b: (0, 0)),
        ],
        out_specs=pl.BlockSpec((1, 1, T), lambda b: (b, 0, 0)),
        out_shape=jax.ShapeDtypeStruct((B, 1, T), jnp.int32),
    )(z, embeddings)


def _sc_gather(table, idx):
    N = idx.shape[0]
    D = table.shape[1]
    NC, NS = 2, 16
    NW = NC * NS
    b_per_w = N // NW
    mesh = plsc.VectorSubcoreMesh(core_axis_name="c", subcore_axis_name="s")

    @functools.partial(
        pl.kernel,
        mesh=mesh,
        out_type=jax.ShapeDtypeStruct((N, D), jnp.float32),
        scratch_types=[
            pltpu.VMEM((b_per_w,), jnp.int32),
            pltpu.VMEM((b_per_w, D), jnp.float32),
            pltpu.SemaphoreType.DMA,
        ],
    )
    def gather_k(table_hbm, idx_hbm, out_hbm, idx_v, rows_v, sem):
        wid = lax.axis_index("s") * NC + lax.axis_index("c")
        base = wid * b_per_w
        pltpu.sync_copy(idx_hbm.at[pl.ds(base, b_per_w)], idx_v)
        pltpu.async_copy(table_hbm.at[idx_v], rows_v, sem).wait()
        pltpu.sync_copy(rows_v, out_hbm.at[pl.ds(base, b_per_w)])

    return gather_k(table, idx)


def kernel(z, embeddings):
    B, D, T = z.shape
    idx = _tc_argmin(z, embeddings).reshape(B * T)
    rows = _sc_gather(embeddings, idx)
    vq = jnp.transpose(rows.reshape(B, T, D), (0, 2, 1))
    return z + (vq - z)


# speedup vs baseline: 1.1912x; 1.1170x over previous
import functools

import jax
import jax.numpy as jnp
from jax import lax
from jax.experimental import pallas as pl
from jax.experimental.pallas import tpu as pltpu
from jax.experimental.pallas import tpu_sc as plsc

K_TOTAL = 8192
D_DIM = 256
K_CHUNK = 1024


def _argmin_body(z_ref, emb_ref, out_ref):
    zb = z_ref[0]
    T = zb.shape[1]
    z2 = jnp.sum(zb * zb, axis=0, keepdims=True)

    ids0 = lax.broadcasted_iota(jnp.int32, (K_CHUNK, T), 0).astype(jnp.float32)

    def step(c, carry):
        minv, mini = carry
        ech2 = emb_ref[pl.ds(c * K_CHUNK, K_CHUNK), :] * 2.0
        e2 = 0.25 * jnp.sum(ech2 * ech2, axis=1, keepdims=True)
        m2 = jnp.dot(ech2, zb, preferred_element_type=jnp.float32)
        d = (z2 + e2) - m2
        cmin = jnp.min(d, axis=0, keepdims=True)
        ids = ids0 + (c * K_CHUNK).astype(jnp.float32)
        cidx = jnp.min(
            jnp.where(d == cmin, ids, jnp.float32(2**30)), axis=0, keepdims=True
        )
        better = cmin < minv
        return (jnp.where(better, cmin, minv), jnp.where(better, cidx, mini))

    minv0 = jnp.full((1, T), jnp.inf, dtype=jnp.float32)
    mini0 = jnp.zeros((1, T), dtype=jnp.float32)
    _, mini = lax.fori_loop(0, K_TOTAL // K_CHUNK, step, (minv0, mini0))
    out_ref[0] = mini.astype(jnp.int32)


def _tc_argmin(z, embeddings):
    B, D, T = z.shape
    return pl.pallas_call(
        _argmin_body,
        grid=(B,),
        in_specs=[
            pl.BlockSpec((1, D, T), lambda b: (b, 0, 0)),
            pl.BlockSpec((K_TOTAL, D), lambda b: (0, 0)),
        ],
        out_specs=pl.BlockSpec((1, 1, T), lambda b: (b, 0, 0)),
        out_shape=jax.ShapeDtypeStruct((B, 1, T), jnp.int32),
    )(z, embeddings)


def _sc_gather(table, idx):
    N = idx.shape[0]
    D = table.shape[1]
    NC, NS = 2, 16
    NW = NC * NS
    b_per_w = N // NW
    mesh = plsc.VectorSubcoreMesh(core_axis_name="c", subcore_axis_name="s")

    @functools.partial(
        pl.kernel,
        mesh=mesh,
        out_type=jax.ShapeDtypeStruct((N, D), jnp.float32),
        scratch_types=[
            pltpu.VMEM((b_per_w,), jnp.int32),
            pltpu.VMEM((b_per_w, D), jnp.float32),
            pltpu.SemaphoreType.DMA,
        ],
    )
    def gather_k(table_hbm, idx_hbm, out_hbm, idx_v, rows_v, sem):
        wid = lax.axis_index("s") * NC + lax.axis_index("c")
        base = wid * b_per_w
        pltpu.sync_copy(idx_hbm.at[pl.ds(base, b_per_w)], idx_v)
        pltpu.async_copy(table_hbm.at[idx_v], rows_v, sem).wait()
        pltpu.sync_copy(rows_v, out_hbm.at[pl.ds(base, b_per_w)])

    return gather_k(table, idx)


def kernel(z, embeddings):
    B, D, T = z.shape
    idx = _tc_argmin(z, embeddings).reshape(B * T)
    rows = _sc_gather(embeddings, idx)
    vq = jnp.transpose(rows.reshape(B, T, D), (0, 2, 1))
    return z + (vq - z)
